# baseline (speedup 1.0000x reference)
import math

import jax
import jax.numpy as jnp
from jax.experimental import pallas as pl
from jax.experimental.pallas import tpu as pltpu

_NB = 32
_H = 16
_N = 2048
_DW = 4224
_NT = _N // 128
_LOG_DENOM = math.log(128 / 8)


def _diag_values(table_ref, h):
    j = jax.lax.broadcasted_iota(jnp.int32, (1, _DW), 1)
    rel = j - (_N - 1)
    neg = -rel
    res = jnp.where(neg < 0, _NB // 2, 0).astype(jnp.int32)
    na = jnp.abs(neg)
    is_small = na < 8
    n_safe = jnp.maximum(na, 1).astype(jnp.float32)
    vil = 8 + (jnp.log(n_safe / 8) / _LOG_DENOM * 8).astype(jnp.int32)
    vil = jnp.minimum(vil, 15)
    bucket = res + jnp.where(is_small, na, vil)
    acc = jnp.zeros((1, _DW), jnp.float32)
    for b in range(_NB):
        acc = jnp.where(bucket == b, table_ref[h, 0, b], acc)
    return acc


def _chunk_copy(shear_ref, out_ref, h, t, sem):
    a = 2048 - 128 * t
    return pltpu.make_async_copy(
        shear_ref.at[:, pl.ds(a, _N)],
        out_ref.at[h, pl.ds(128 * t, 128), :],
        sem,
    )


_NBUF = 4


def _bias_body(table_ref, out_ref, *scratch):
    shears = scratch[:_NBUF]
    sems = scratch[_NBUF:]
    for h in range(_H):
        sh, sem = shears[h % _NBUF], sems[h % _NBUF]
        if h >= _NBUF:
            for t in range(_NT):
                _chunk_copy(sh, out_ref, h - _NBUF, t, sem).wait()
        diag = _diag_values(table_ref, h)
        rep = jnp.broadcast_to(diag, (128, _DW))
        sh[...] = pltpu.roll(rep, 1, 1, stride=1, stride_axis=0)
        for t in range(_NT):
            _chunk_copy(sh, out_ref, h, t, sem).start()
    for h in range(_H - _NBUF, _H):
        for t in range(_NT):
            _chunk_copy(shears[h % _NBUF], out_ref, h, t, sems[h % _NBUF]).wait()


@jax.jit
def _rpb(table_t):
    return pl.pallas_call(
        _bias_body,
        in_specs=[pl.BlockSpec(memory_space=pltpu.VMEM)],
        out_specs=pl.BlockSpec(memory_space=pl.ANY),
        out_shape=jax.ShapeDtypeStruct((_H, _N, _N), jnp.float32),
        scratch_shapes=(
            [pltpu.VMEM((128, _DW), jnp.float32)] * _NBUF
            + [pltpu.SemaphoreType.DMA] * _NBUF
        ),
    )(table_t)


def kernel(n, rel_bias_table):
    del n
    table_t = rel_bias_table.T.reshape(_H, 1, _NB)
    return _rpb(table_t)


# speedup vs baseline: 1.0219x; 1.0219x over previous
import math

import jax
import jax.numpy as jnp
from jax.experimental import pallas as pl
from jax.experimental.pallas import tpu as pltpu

_NB = 32
_H = 16
_N = 2048
_DW = 4224
_NT = _N // 128
_LOG_DENOM = math.log(128 / 8)


def _diag_values(table_ref, h):
    j = jax.lax.broadcasted_iota(jnp.int32, (1, _DW), 1)
    rel = j - (_N - 1)
    neg = -rel
    res = jnp.where(neg < 0, _NB // 2, 0).astype(jnp.int32)
    na = jnp.abs(neg)
    is_small = na < 8
    n_safe = jnp.maximum(na, 1).astype(jnp.float32)
    vil = 8 + (jnp.log(n_safe / 8) / _LOG_DENOM * 8).astype(jnp.int32)
    vil = jnp.minimum(vil, 15)
    bucket = res + jnp.where(is_small, na, vil)
    acc = jnp.zeros((1, _DW), jnp.float32)
    for b in range(_NB):
        acc = jnp.where(bucket == b, table_ref[h, 0, b], acc)
    return acc


def _chunk_copy(shear_ref, out_ref, h, t, sem):
    a = 2048 - 128 * t
    return pltpu.make_async_copy(
        shear_ref.at[:, pl.ds(a, _N)],
        out_ref.at[h, pl.ds(128 * t, 128), :],
        sem,
    )


_NBUF = 2


def _bias_body(table_ref, out_ref, *scratch):
    shears = scratch[:_NBUF]
    sems = scratch[_NBUF:]
    for h in range(_H):
        sh, sem = shears[h % _NBUF], sems[h % _NBUF]
        if h >= _NBUF:
            for t in range(_NT):
                _chunk_copy(sh, out_ref, h - _NBUF, t, sem).wait()
        diag = _diag_values(table_ref, h)
        rep = jnp.broadcast_to(diag, (128, _DW))
        sh[...] = pltpu.roll(rep, 1, 1, stride=1, stride_axis=0)
        for t in range(_NT):
            _chunk_copy(sh, out_ref, h, t, sem).start()
    for h in range(_H - _NBUF, _H):
        for t in range(_NT):
            _chunk_copy(shears[h % _NBUF], out_ref, h, t, sems[h % _NBUF]).wait()


@jax.jit
def _rpb(table_t):
    return pl.pallas_call(
        _bias_body,
        in_specs=[pl.BlockSpec(memory_space=pltpu.VMEM)],
        out_specs=pl.BlockSpec(memory_space=pl.ANY),
        out_shape=jax.ShapeDtypeStruct((_H, _N, _N), jnp.float32),
        scratch_shapes=(
            [pltpu.VMEM((128, _DW), jnp.float32)] * _NBUF
            + [pltpu.SemaphoreType.DMA] * _NBUF
        ),
    )(table_t)


def kernel(n, rel_bias_table):
    del n
    table_t = rel_bias_table.T.reshape(_H, 1, _NB)
    return _rpb(table_t)
